# baseline (speedup 1.0000x reference)
import functools

import jax
import jax.numpy as jnp
from jax import lax
from jax.experimental import pallas as pl
from jax.experimental.pallas import tpu as pltpu
from jax.experimental.pallas import tpu_sc as plsc

N = 20000
NP = 20480
M = 5000
D = 128
MP = 5120
BN = 2048
BM = 1024

NC = 2
NS = 16
NW = NC * NS
BPW = MP // NW
HALF = BPW // 2


def _argmin_body(v_ref, a2_ref, s_ref, idx_ref, minv_ref, mini_ref):
    m = pl.program_id(0)
    n = pl.program_id(1)

    a2 = a2_ref[...]
    st = s_ref[...]
    b2 = st[0:1] * st[0:1] + st[1:2] * st[1:2] + st[2:3] * st[2:3]
    ab2 = jnp.dot(v_ref[...], st)
    d2 = (a2 + b2) - ab2

    bmin = jnp.min(d2, axis=0, keepdims=True)
    bminc = jnp.maximum(bmin, 0.0)
    s = jnp.sqrt(bminc)
    c = s * s
    cb = lax.bitcast_convert_type(c, jnp.int32)
    T = bminc
    for k in range(-4, 5):
        cand = lax.bitcast_convert_type(cb + k, jnp.float32)
        ok = jnp.sqrt(cand) <= s
        T = jnp.where(ok, jnp.maximum(T, cand), T)
    T = jnp.where(s == 0.0, 0.0, T)

    rows = lax.broadcasted_iota(jnp.int32, (BN, BM), 0)
    bidx = jnp.min(
        jnp.where(d2 <= T, rows, jnp.int32(2**30)),
        axis=0, keepdims=True) + n * BN

    @pl.when(n == 0)
    def _():
        minv_ref[...] = s
        mini_ref[...] = bidx

    @pl.when(n > 0)
    def _():
        rv = minv_ref[...]
        ri = mini_ref[...]
        better = s < rv
        minv_ref[...] = jnp.where(better, s, rv)
        mini_ref[...] = jnp.where(better, bidx, ri)

    @pl.when(n == NP // BN - 1)
    def _():
        idx_ref[...] = mini_ref[...]


def _nearest_idx(vertices, a2, s_t):
    return pl.pallas_call(
        _argmin_body,
        grid=(MP // BM, NP // BN),
        in_specs=[
            pl.BlockSpec((BN, 3), lambda m, n: (n, 0)),
            pl.BlockSpec((BN, 1), lambda m, n: (n, 0)),
            pl.BlockSpec((3, BM), lambda m, n: (0, m)),
        ],
        out_specs=pl.BlockSpec((1, BM), lambda m, n: (0, m)),
        out_shape=jax.ShapeDtypeStruct((1, MP), jnp.int32),
        scratch_shapes=[
            pltpu.VMEM((1, BM), jnp.float32),
            pltpu.VMEM((1, BM), jnp.int32),
        ],
    )(vertices, a2, s_t)


@functools.lru_cache(maxsize=None)
def _make_sc_gather():
    @functools.partial(
        pl.kernel,
        mesh=plsc.VectorSubcoreMesh(core_axis_name="c", subcore_axis_name="s"),
        out_type=jax.ShapeDtypeStruct((MP, D), jnp.float32),
        scratch_types=[
            pltpu.VMEM((2, HALF), jnp.int32),
            pltpu.VMEM((BPW, D), jnp.float32),
            pltpu.SemaphoreType.DMA,
        ],
    )
    def _sc_gather(x_hbm, idx_hbm, out_hbm, idx_v, rows_v, sem):
        wid = lax.axis_index("s") * NC + lax.axis_index("c")
        for j in range(2):
            pltpu.sync_copy(idx_hbm.at[2 * wid + j], idx_v.at[j])
            pltpu.async_copy(
                x_hbm.at[idx_v.at[j]], rows_v.at[pl.ds(j * HALF, HALF)], sem
            ).wait()
        pltpu.sync_copy(rows_v, out_hbm.at[pl.ds(wid * BPW, BPW)])

    return _sc_gather


def kernel(vertices, sub_vertices, X):
    s_t = jnp.zeros((3, MP), jnp.float32).at[:, :M].set(sub_vertices.T)
    v2 = jnp.full((NP, 3), 1.0e18, jnp.float32).at[:N].set(2.0 * vertices)
    a2 = 0.25 * jnp.sum(v2 * v2, axis=1, keepdims=True)
    idx = _nearest_idx(v2, a2, s_t)
    idx2 = idx.reshape(NW * 2, HALF)
    rows = _make_sc_gather()(X, idx2)
    return rows[:M]


# speedup vs baseline: 1.0315x; 1.0315x over previous
import functools

import jax
import jax.numpy as jnp
from jax import lax
from jax.experimental import pallas as pl
from jax.experimental.pallas import tpu as pltpu
from jax.experimental.pallas import tpu_sc as plsc

N = 20000
NP = 20480
M = 5000
D = 128
MP = 5120
BN = 4096
BM = 1024

NC = 2
NS = 16
NW = NC * NS
BPW = MP // NW
HALF = BPW // 2


def _argmin_body(v_ref, a2_ref, s_ref, idx_ref, minv_ref, mini_ref):
    m = pl.program_id(0)
    n = pl.program_id(1)

    a2 = a2_ref[...]
    st = s_ref[...]
    b2 = st[0:1] * st[0:1] + st[1:2] * st[1:2] + st[2:3] * st[2:3]
    ab2 = jnp.dot(v_ref[...], st)
    d2 = (a2 + b2) - ab2

    bmin = jnp.min(d2, axis=0, keepdims=True)
    bminc = jnp.maximum(bmin, 0.0)
    s = jnp.sqrt(bminc)
    c = s * s
    cb = lax.bitcast_convert_type(c, jnp.int32)
    T = bminc
    for k in range(-4, 5):
        cand = lax.bitcast_convert_type(cb + k, jnp.float32)
        ok = jnp.sqrt(cand) <= s
        T = jnp.where(ok, jnp.maximum(T, cand), T)
    T = jnp.where(s == 0.0, 0.0, T)

    rows = lax.broadcasted_iota(jnp.int32, (BN, BM), 0)
    bidx = jnp.min(
        jnp.where(d2 <= T, rows, jnp.int32(2**30)),
        axis=0, keepdims=True) + n * BN

    @pl.when(n == 0)
    def _():
        minv_ref[...] = s
        mini_ref[...] = bidx

    @pl.when(n > 0)
    def _():
        rv = minv_ref[...]
        ri = mini_ref[...]
        better = s < rv
        minv_ref[...] = jnp.where(better, s, rv)
        mini_ref[...] = jnp.where(better, bidx, ri)

    @pl.when(n == NP // BN - 1)
    def _():
        idx_ref[...] = mini_ref[...]


def _nearest_idx(vertices, a2, s_t):
    return pl.pallas_call(
        _argmin_body,
        grid=(MP // BM, NP // BN),
        in_specs=[
            pl.BlockSpec((BN, 3), lambda m, n: (n, 0)),
            pl.BlockSpec((BN, 1), lambda m, n: (n, 0)),
            pl.BlockSpec((3, BM), lambda m, n: (0, m)),
        ],
        out_specs=pl.BlockSpec((1, BM), lambda m, n: (0, m)),
        out_shape=jax.ShapeDtypeStruct((1, MP), jnp.int32),
        scratch_shapes=[
            pltpu.VMEM((1, BM), jnp.float32),
            pltpu.VMEM((1, BM), jnp.int32),
        ],
    )(vertices, a2, s_t)


@functools.lru_cache(maxsize=None)
def _make_sc_gather():
    @functools.partial(
        pl.kernel,
        mesh=plsc.VectorSubcoreMesh(core_axis_name="c", subcore_axis_name="s"),
        out_type=jax.ShapeDtypeStruct((MP, D), jnp.float32),
        scratch_types=[
            pltpu.VMEM((2, HALF), jnp.int32),
            pltpu.VMEM((BPW, D), jnp.float32),
            pltpu.SemaphoreType.DMA,
        ],
    )
    def _sc_gather(x_hbm, idx_hbm, out_hbm, idx_v, rows_v, sem):
        wid = lax.axis_index("s") * NC + lax.axis_index("c")
        for j in range(2):
            pltpu.sync_copy(idx_hbm.at[2 * wid + j], idx_v.at[j])
            pltpu.async_copy(
                x_hbm.at[idx_v.at[j]], rows_v.at[pl.ds(j * HALF, HALF)], sem
            ).wait()
        pltpu.sync_copy(rows_v, out_hbm.at[pl.ds(wid * BPW, BPW)])

    return _sc_gather


def kernel(vertices, sub_vertices, X):
    s_t = jnp.zeros((3, MP), jnp.float32).at[:, :M].set(sub_vertices.T)
    v2 = jnp.full((NP, 3), 1.0e18, jnp.float32).at[:N].set(2.0 * vertices)
    a2 = 0.25 * jnp.sum(v2 * v2, axis=1, keepdims=True)
    idx = _nearest_idx(v2, a2, s_t)
    idx2 = idx.reshape(NW * 2, HALF)
    rows = _make_sc_gather()(X, idx2)
    return rows[:M]


# speedup vs baseline: 1.1092x; 1.0754x over previous
import functools

import jax
import jax.numpy as jnp
from jax import lax
from jax.experimental import pallas as pl
from jax.experimental.pallas import tpu as pltpu
from jax.experimental.pallas import tpu_sc as plsc

N = 20000
NP = 20480
M = 5000
D = 128
MP = 5120
BN = 4096
BM = 2560

NC = 2
NS = 16
NW = NC * NS
BPW = MP // NW
HALF = BPW // 2


def _argmin_body(v_ref, a2_ref, s_ref, idx_ref, minv_ref, mini_ref):
    m = pl.program_id(0)
    n = pl.program_id(1)

    a2 = a2_ref[...]
    st = s_ref[...]
    b2 = st[0:1] * st[0:1] + st[1:2] * st[1:2] + st[2:3] * st[2:3]
    ab2 = jnp.dot(v_ref[...], st)
    d2 = (a2 + b2) - ab2

    bmin = jnp.min(d2, axis=0, keepdims=True)
    bminc = jnp.maximum(bmin, 0.0)
    s = jnp.sqrt(bminc)
    c = s * s
    cb = lax.bitcast_convert_type(c, jnp.int32)
    T = bminc
    for k in range(-4, 5):
        cand = lax.bitcast_convert_type(cb + k, jnp.float32)
        ok = jnp.sqrt(cand) <= s
        T = jnp.where(ok, jnp.maximum(T, cand), T)
    T = jnp.where(s == 0.0, 0.0, T)

    rows = lax.broadcasted_iota(jnp.int32, (BN, BM), 0)
    bidx = jnp.min(
        jnp.where(d2 <= T, rows, jnp.int32(2**30)),
        axis=0, keepdims=True) + n * BN

    @pl.when(n == 0)
    def _():
        minv_ref[...] = s
        mini_ref[...] = bidx

    @pl.when(n > 0)
    def _():
        rv = minv_ref[...]
        ri = mini_ref[...]
        better = s < rv
        minv_ref[...] = jnp.where(better, s, rv)
        mini_ref[...] = jnp.where(better, bidx, ri)

    @pl.when(n == NP // BN - 1)
    def _():
        idx_ref[...] = mini_ref[...]


def _nearest_idx(vertices, a2, s_t):
    return pl.pallas_call(
        _argmin_body,
        grid=(MP // BM, NP // BN),
        in_specs=[
            pl.BlockSpec((BN, 3), lambda m, n: (n, 0)),
            pl.BlockSpec((BN, 1), lambda m, n: (n, 0)),
            pl.BlockSpec((3, BM), lambda m, n: (0, m)),
        ],
        out_specs=pl.BlockSpec((1, BM), lambda m, n: (0, m)),
        out_shape=jax.ShapeDtypeStruct((1, MP), jnp.int32),
        scratch_shapes=[
            pltpu.VMEM((1, BM), jnp.float32),
            pltpu.VMEM((1, BM), jnp.int32),
        ],
    )(vertices, a2, s_t)


@functools.lru_cache(maxsize=None)
def _make_sc_gather():
    @functools.partial(
        pl.kernel,
        mesh=plsc.VectorSubcoreMesh(core_axis_name="c", subcore_axis_name="s"),
        out_type=jax.ShapeDtypeStruct((MP, D), jnp.float32),
        scratch_types=[
            pltpu.VMEM((2, HALF), jnp.int32),
            pltpu.VMEM((BPW, D), jnp.float32),
            pltpu.SemaphoreType.DMA,
        ],
    )
    def _sc_gather(x_hbm, idx_hbm, out_hbm, idx_v, rows_v, sem):
        wid = lax.axis_index("s") * NC + lax.axis_index("c")
        for j in range(2):
            pltpu.sync_copy(idx_hbm.at[2 * wid + j], idx_v.at[j])
            pltpu.async_copy(
                x_hbm.at[idx_v.at[j]], rows_v.at[pl.ds(j * HALF, HALF)], sem
            ).wait()
        pltpu.sync_copy(rows_v, out_hbm.at[pl.ds(wid * BPW, BPW)])

    return _sc_gather


def kernel(vertices, sub_vertices, X):
    s_t = jnp.zeros((3, MP), jnp.float32).at[:, :M].set(sub_vertices.T)
    v2 = jnp.full((NP, 3), 1.0e18, jnp.float32).at[:N].set(2.0 * vertices)
    a2 = 0.25 * jnp.sum(v2 * v2, axis=1, keepdims=True)
    idx = _nearest_idx(v2, a2, s_t)
    idx2 = idx.reshape(NW * 2, HALF)
    rows = _make_sc_gather()(X, idx2)
    return rows[:M]


# speedup vs baseline: 1.1178x; 1.0077x over previous
import functools

import jax
import jax.numpy as jnp
from jax import lax
from jax.experimental import pallas as pl
from jax.experimental.pallas import tpu as pltpu
from jax.experimental.pallas import tpu_sc as plsc

N = 20000
NP = 20480
M = 5000
D = 128
MP = 5120
BN = 2048
BM = 5120

NC = 2
NS = 16
NW = NC * NS
BPW = MP // NW
HALF = BPW // 2


def _argmin_body(v_ref, a2_ref, s_ref, idx_ref, minv_ref, mini_ref):
    m = pl.program_id(0)
    n = pl.program_id(1)

    a2 = a2_ref[...]
    st = s_ref[...]
    b2 = st[0:1] * st[0:1] + st[1:2] * st[1:2] + st[2:3] * st[2:3]
    ab2 = jnp.dot(v_ref[...], st)
    d2 = (a2 + b2) - ab2

    bmin = jnp.min(d2, axis=0, keepdims=True)
    bminc = jnp.maximum(bmin, 0.0)
    s = jnp.sqrt(bminc)
    c = s * s
    cb = lax.bitcast_convert_type(c, jnp.int32)
    T = bminc
    for k in range(-4, 5):
        cand = lax.bitcast_convert_type(cb + k, jnp.float32)
        ok = jnp.sqrt(cand) <= s
        T = jnp.where(ok, jnp.maximum(T, cand), T)
    T = jnp.where(s == 0.0, 0.0, T)

    rows = lax.broadcasted_iota(jnp.int32, (BN, BM), 0)
    bidx = jnp.min(
        jnp.where(d2 <= T, rows, jnp.int32(2**30)),
        axis=0, keepdims=True) + n * BN

    @pl.when(n == 0)
    def _():
        minv_ref[...] = s
        mini_ref[...] = bidx

    @pl.when(n > 0)
    def _():
        rv = minv_ref[...]
        ri = mini_ref[...]
        better = s < rv
        minv_ref[...] = jnp.where(better, s, rv)
        mini_ref[...] = jnp.where(better, bidx, ri)

    @pl.when(n == NP // BN - 1)
    def _():
        idx_ref[...] = mini_ref[...]


def _nearest_idx(vertices, a2, s_t):
    return pl.pallas_call(
        _argmin_body,
        grid=(MP // BM, NP // BN),
        in_specs=[
            pl.BlockSpec((BN, 3), lambda m, n: (n, 0)),
            pl.BlockSpec((BN, 1), lambda m, n: (n, 0)),
            pl.BlockSpec((3, BM), lambda m, n: (0, m)),
        ],
        out_specs=pl.BlockSpec((1, BM), lambda m, n: (0, m)),
        out_shape=jax.ShapeDtypeStruct((1, MP), jnp.int32),
        scratch_shapes=[
            pltpu.VMEM((1, BM), jnp.float32),
            pltpu.VMEM((1, BM), jnp.int32),
        ],
    )(vertices, a2, s_t)


@functools.lru_cache(maxsize=None)
def _make_sc_gather():
    @functools.partial(
        pl.kernel,
        mesh=plsc.VectorSubcoreMesh(core_axis_name="c", subcore_axis_name="s"),
        out_type=jax.ShapeDtypeStruct((MP, D), jnp.float32),
        scratch_types=[
            pltpu.VMEM((2, HALF), jnp.int32),
            pltpu.VMEM((BPW, D), jnp.float32),
            pltpu.SemaphoreType.DMA,
        ],
    )
    def _sc_gather(x_hbm, idx_hbm, out_hbm, idx_v, rows_v, sem):
        wid = lax.axis_index("s") * NC + lax.axis_index("c")
        for j in range(2):
            pltpu.sync_copy(idx_hbm.at[2 * wid + j], idx_v.at[j])
            pltpu.async_copy(
                x_hbm.at[idx_v.at[j]], rows_v.at[pl.ds(j * HALF, HALF)], sem
            ).wait()
        pltpu.sync_copy(rows_v, out_hbm.at[pl.ds(wid * BPW, BPW)])

    return _sc_gather


def kernel(vertices, sub_vertices, X):
    s_t = jnp.zeros((3, MP), jnp.float32).at[:, :M].set(sub_vertices.T)
    v2 = jnp.full((NP, 3), 1.0e18, jnp.float32).at[:N].set(2.0 * vertices)
    a2 = 0.25 * jnp.sum(v2 * v2, axis=1, keepdims=True)
    idx = _nearest_idx(v2, a2, s_t)
    idx2 = idx.reshape(NW * 2, HALF)
    rows = _make_sc_gather()(X, idx2)
    return rows[:M]


# speedup vs baseline: 1.1248x; 1.0062x over previous
import functools

import jax
import jax.numpy as jnp
from jax import lax
from jax.experimental import pallas as pl
from jax.experimental.pallas import tpu as pltpu
from jax.experimental.pallas import tpu_sc as plsc

N = 20000
NP = 20480
M = 5000
D = 128
MP = 5120
BN = 2048
BM = 5120

NC = 2
NS = 16
NW = NC * NS
BPW = MP // NW
HALF = BPW // 2


def _argmin_body(v_ref, a2_ref, s_ref, idx_ref, minv_ref, mini_ref):
    m = pl.program_id(0)
    n = pl.program_id(1)

    a2 = a2_ref[...]
    st = s_ref[...]
    b2 = st[0:1] * st[0:1] + st[1:2] * st[1:2] + st[2:3] * st[2:3]
    ab2 = jnp.dot(v_ref[...], st)
    d2 = (a2 + b2) - ab2

    bmin = jnp.min(d2, axis=0, keepdims=True)
    bminc = jnp.maximum(bmin, 0.0)
    s = jnp.sqrt(bminc)
    c = s * s
    cb = lax.bitcast_convert_type(c, jnp.int32)
    T = bminc
    for k in range(-4, 5):
        cand = lax.bitcast_convert_type(cb + k, jnp.float32)
        ok = jnp.sqrt(cand) <= s
        T = jnp.where(ok, jnp.maximum(T, cand), T)
    T = jnp.where(s == 0.0, 0.0, T)

    rows = lax.broadcasted_iota(jnp.int32, (BN, BM), 0)
    bidx = jnp.min(
        jnp.where(d2 <= T, rows, jnp.int32(2**30)),
        axis=0, keepdims=True) + n * BN

    @pl.when(n == 0)
    def _():
        minv_ref[...] = s
        mini_ref[...] = bidx

    @pl.when(n > 0)
    def _():
        rv = minv_ref[...]
        ri = mini_ref[...]
        better = s < rv
        minv_ref[...] = jnp.where(better, s, rv)
        mini_ref[...] = jnp.where(better, bidx, ri)

    @pl.when(n == NP // BN - 1)
    def _():
        idx_ref[...] = mini_ref[...]


def _nearest_idx(vertices, a2, s_t):
    return pl.pallas_call(
        _argmin_body,
        grid=(MP // BM, NP // BN),
        in_specs=[
            pl.BlockSpec((BN, 3), lambda m, n: (n, 0)),
            pl.BlockSpec((BN, 1), lambda m, n: (n, 0)),
            pl.BlockSpec((3, BM), lambda m, n: (0, m)),
        ],
        out_specs=pl.BlockSpec((1, BM), lambda m, n: (0, m)),
        out_shape=jax.ShapeDtypeStruct((1, MP), jnp.int32),
        scratch_shapes=[
            pltpu.VMEM((1, BM), jnp.float32),
            pltpu.VMEM((1, BM), jnp.int32),
        ],
    )(vertices, a2, s_t)


def _prep_body(vert_ref, sub_ref, v2_ref, a2_ref, st_ref):
    two_v = 2.0 * vert_ref[...]
    v2_ref[pl.ds(0, N), :] = two_v
    v2_ref[pl.ds(N, NP - N), :] = jnp.full((NP - N, 3), 1.0e18, jnp.float32)
    x, y, z = two_v[:, 0:1], two_v[:, 1:2], two_v[:, 2:3]
    a2_ref[pl.ds(0, N), :] = 0.25 * (x * x + y * y + z * z)
    a2_ref[pl.ds(N, NP - N), :] = jnp.full((NP - N, 1), 1.0e30, jnp.float32)
    st_ref[:, pl.ds(0, M)] = jnp.transpose(sub_ref[...])
    st_ref[:, pl.ds(M, MP - M)] = jnp.zeros((3, MP - M), jnp.float32)


def _prep(vertices, sub_vertices):
    return pl.pallas_call(
        _prep_body,
        out_shape=[
            jax.ShapeDtypeStruct((NP, 3), jnp.float32),
            jax.ShapeDtypeStruct((NP, 1), jnp.float32),
            jax.ShapeDtypeStruct((3, MP), jnp.float32),
        ],
    )(vertices, sub_vertices)


@functools.lru_cache(maxsize=None)
def _make_sc_gather():
    @functools.partial(
        pl.kernel,
        mesh=plsc.VectorSubcoreMesh(core_axis_name="c", subcore_axis_name="s"),
        out_type=jax.ShapeDtypeStruct((M, D), jnp.float32),
        scratch_types=[
            pltpu.VMEM((2, HALF), jnp.int32),
            pltpu.VMEM((BPW, D), jnp.float32),
            pltpu.SemaphoreType.DMA,
        ],
    )
    def _sc_gather(x_hbm, idx_hbm, out_hbm, idx_v, rows_v, sem):
        wid = lax.axis_index("s") * NC + lax.axis_index("c")
        for j in range(2):
            pltpu.sync_copy(idx_hbm.at[2 * wid + j], idx_v.at[j])
            pltpu.async_copy(
                x_hbm.at[idx_v.at[j]], rows_v.at[pl.ds(j * HALF, HALF)], sem
            ).wait()

        @pl.when(wid < NW - 1)
        def _():
            pltpu.sync_copy(rows_v, out_hbm.at[pl.ds(wid * BPW, BPW)])

        @pl.when(wid == NW - 1)
        def _():
            tail = M - (NW - 1) * BPW
            pltpu.sync_copy(
                rows_v.at[pl.ds(0, tail)],
                out_hbm.at[pl.ds((NW - 1) * BPW, tail)],
            )

    return _sc_gather


def kernel(vertices, sub_vertices, X):
    v2, a2, s_t = _prep(vertices, sub_vertices)
    idx = _nearest_idx(v2, a2, s_t)
    idx2 = idx.reshape(NW * 2, HALF)
    return _make_sc_gather()(X, idx2)


# speedup vs baseline: 1.1351x; 1.0092x over previous
import functools

import jax
import jax.numpy as jnp
from jax import lax
from jax.experimental import pallas as pl
from jax.experimental.pallas import tpu as pltpu
from jax.experimental.pallas import tpu_sc as plsc

N = 20000
NP = 20480
M = 5000
D = 128
MP = 5120
BN = 2560
BM = 5120

NC = 2
NS = 16
NW = NC * NS
BPW = MP // NW
HALF = BPW // 2


def _argmin_body(v_ref, a2_ref, s_ref, idx_ref, minv_ref, mini_ref):
    m = pl.program_id(0)
    n = pl.program_id(1)

    a2 = a2_ref[...]
    st = s_ref[...]
    b2 = st[0:1] * st[0:1] + st[1:2] * st[1:2] + st[2:3] * st[2:3]
    ab2 = jnp.dot(v_ref[...], st)
    d2 = (a2 + b2) - ab2

    bmin = jnp.min(d2, axis=0, keepdims=True)
    bminc = jnp.maximum(bmin, 0.0)
    s = jnp.sqrt(bminc)
    c = s * s
    cb = lax.bitcast_convert_type(c, jnp.int32)
    T = bminc
    for k in range(-4, 5):
        cand = lax.bitcast_convert_type(cb + k, jnp.float32)
        ok = jnp.sqrt(cand) <= s
        T = jnp.where(ok, jnp.maximum(T, cand), T)
    T = jnp.where(s == 0.0, 0.0, T)

    rows = lax.broadcasted_iota(jnp.int32, (BN, BM), 0)
    bidx = jnp.min(
        jnp.where(d2 <= T, rows, jnp.int32(2**30)),
        axis=0, keepdims=True) + n * BN

    @pl.when(n == 0)
    def _():
        minv_ref[...] = s
        mini_ref[...] = bidx

    @pl.when(n > 0)
    def _():
        rv = minv_ref[...]
        ri = mini_ref[...]
        better = s < rv
        minv_ref[...] = jnp.where(better, s, rv)
        mini_ref[...] = jnp.where(better, bidx, ri)

    @pl.when(n == NP // BN - 1)
    def _():
        idx_ref[...] = mini_ref[...]


def _nearest_idx(vertices, a2, s_t):
    return pl.pallas_call(
        _argmin_body,
        grid=(MP // BM, NP // BN),
        in_specs=[
            pl.BlockSpec((BN, 3), lambda m, n: (n, 0)),
            pl.BlockSpec((BN, 1), lambda m, n: (n, 0)),
            pl.BlockSpec((3, BM), lambda m, n: (0, m)),
        ],
        out_specs=pl.BlockSpec((1, BM), lambda m, n: (0, m)),
        out_shape=jax.ShapeDtypeStruct((1, MP), jnp.int32),
        scratch_shapes=[
            pltpu.VMEM((1, BM), jnp.float32),
            pltpu.VMEM((1, BM), jnp.int32),
        ],
    )(vertices, a2, s_t)


def _prep_body(vert_ref, sub_ref, v2_ref, a2_ref, st_ref):
    two_v = 2.0 * vert_ref[...]
    v2_ref[pl.ds(0, N), :] = two_v
    v2_ref[pl.ds(N, NP - N), :] = jnp.full((NP - N, 3), 1.0e18, jnp.float32)
    x, y, z = two_v[:, 0:1], two_v[:, 1:2], two_v[:, 2:3]
    a2_ref[pl.ds(0, N), :] = 0.25 * (x * x + y * y + z * z)
    a2_ref[pl.ds(N, NP - N), :] = jnp.full((NP - N, 1), 1.0e30, jnp.float32)
    st_ref[:, pl.ds(0, M)] = jnp.transpose(sub_ref[...])
    st_ref[:, pl.ds(M, MP - M)] = jnp.zeros((3, MP - M), jnp.float32)


def _prep(vertices, sub_vertices):
    return pl.pallas_call(
        _prep_body,
        out_shape=[
            jax.ShapeDtypeStruct((NP, 3), jnp.float32),
            jax.ShapeDtypeStruct((NP, 1), jnp.float32),
            jax.ShapeDtypeStruct((3, MP), jnp.float32),
        ],
    )(vertices, sub_vertices)


@functools.lru_cache(maxsize=None)
def _make_sc_gather():
    @functools.partial(
        pl.kernel,
        mesh=plsc.VectorSubcoreMesh(core_axis_name="c", subcore_axis_name="s"),
        out_type=jax.ShapeDtypeStruct((M, D), jnp.float32),
        scratch_types=[
            pltpu.VMEM((2, HALF), jnp.int32),
            pltpu.VMEM((BPW, D), jnp.float32),
            pltpu.SemaphoreType.DMA,
        ],
    )
    def _sc_gather(x_hbm, idx_hbm, out_hbm, idx_v, rows_v, sem):
        wid = lax.axis_index("s") * NC + lax.axis_index("c")
        for j in range(2):
            pltpu.sync_copy(idx_hbm.at[2 * wid + j], idx_v.at[j])
            pltpu.async_copy(
                x_hbm.at[idx_v.at[j]], rows_v.at[pl.ds(j * HALF, HALF)], sem
            ).wait()

        @pl.when(wid < NW - 1)
        def _():
            pltpu.sync_copy(rows_v, out_hbm.at[pl.ds(wid * BPW, BPW)])

        @pl.when(wid == NW - 1)
        def _():
            tail = M - (NW - 1) * BPW
            pltpu.sync_copy(
                rows_v.at[pl.ds(0, tail)],
                out_hbm.at[pl.ds((NW - 1) * BPW, tail)],
            )

    return _sc_gather


def kernel(vertices, sub_vertices, X):
    v2, a2, s_t = _prep(vertices, sub_vertices)
    idx = _nearest_idx(v2, a2, s_t)
    idx2 = idx.reshape(NW * 2, HALF)
    return _make_sc_gather()(X, idx2)
